# baseline (device time: 147356 ns/iter reference)
import jax
import jax.numpy as jnp
from jax import lax
from jax.experimental import pallas as pl
from jax.experimental.pallas import tpu as pltpu

N_DEV = 4
M_PER = 1024
K = 4096
N_PER = 2048
NB = 512
N_T = 4
N_JN = N_PER // NB


def kernel(x, w_mat):
    x = x.astype(jnp.bfloat16)

    def body(x_ref, w_ref, out_ref, send_buf, send_sems,
             recv_sems, local_sem):
        t = pl.program_id(0)
        jn = pl.program_id(1)
        my = lax.axis_index("i")

        @pl.when(jnp.logical_and(t == 0, jn == 0))
        def _():
            bar = pltpu.get_barrier_semaphore()
            for off in (1, 2, 3):
                pl.semaphore_signal(
                    bar, inc=1,
                    device_id=((my + off) % N_DEV,),
                    device_id_type=pl.DeviceIdType.MESH,
                )
            pl.semaphore_wait(bar, 3)

        part = jnp.dot(
            x_ref[...],
            w_ref[...].astype(jnp.bfloat16),
            preferred_element_type=jnp.float32,
        )

        def my_dst(jj):
            return out_ref.at[pl.ds(my * M_PER, M_PER), pl.ds(jj * NB, NB)]

        if True:
            slot = (t % 2) * N_JN + jn

            @pl.when(t >= 2)
            def _():
                pltpu.make_async_remote_copy(
                    src_ref=send_buf.at[slot],
                    dst_ref=my_dst(jn),
                    send_sem=send_sems.at[t - 2, jn],
                    recv_sem=recv_sems.at[0, 0],
                    device_id=(my,),
                    device_id_type=pl.DeviceIdType.MESH,
                ).wait_send()

            y = (part * jax.nn.sigmoid(part)).astype(jnp.bfloat16)
            for st in range(2):
                for sj in range(N_JN):
                    @pl.when(jnp.logical_and(t % 2 == st, jn == sj))
                    def _(st=st, sj=sj):
                        send_buf[st * N_JN + sj] = y

            @pl.when(t < N_T - 1)
            def _():
                d = (my + 1 + t) % N_DEV
                rdma = pltpu.make_async_remote_copy(
                    src_ref=send_buf.at[slot],
                    dst_ref=my_dst(jn),
                    send_sem=send_sems.at[t, jn],
                    recv_sem=recv_sems.at[t, jn],
                    device_id=(d,),
                    device_id_type=pl.DeviceIdType.MESH,
                )
                rdma.start()

            @pl.when(t == N_T - 1)
            def _():
                cp = pltpu.make_async_copy(send_buf.at[slot], my_dst(jn),
                                           local_sem)
                cp.start()

        last = jnp.logical_and(t == N_T - 1, jn == N_JN - 1)

        @pl.when(last)
        def _():
            for jj in range(N_JN):
                pltpu.make_async_remote_copy(
                    src_ref=send_buf.at[jj],
                    dst_ref=my_dst(jj),
                    send_sem=send_sems.at[N_T - 2, jj],
                    recv_sem=recv_sems.at[0, 0],
                    device_id=(my,),
                    device_id_type=pl.DeviceIdType.MESH,
                ).wait_send()
            for tt in range(N_T - 1):
                for jj in range(N_JN):
                    pltpu.make_async_remote_copy(
                        src_ref=send_buf.at[jj],
                        dst_ref=my_dst(jj),
                        send_sem=send_sems.at[tt, jj],
                        recv_sem=recv_sems.at[tt, jj],
                        device_id=(my,),
                        device_id_type=pl.DeviceIdType.MESH,
                    ).wait_recv()
            for jj in range(N_JN):
                pltpu.make_async_copy(
                    send_buf.at[N_JN + jj], my_dst(jj), local_sem,
                ).wait()

    def w_imap(t, jn):
        d = (lax.axis_index("i") + 1 + t) % N_DEV
        return (0, d * N_JN + jn)

    return pl.pallas_call(
        body,
        grid=(N_T, N_JN),
        out_shape=jax.ShapeDtypeStruct((N_DEV * M_PER, N_PER), jnp.bfloat16),
        in_specs=[
            pl.BlockSpec((M_PER, K), lambda t, jn: (0, 0)),
            pl.BlockSpec((K, NB), w_imap),
        ],
        out_specs=pl.BlockSpec(memory_space=pl.ANY),
        scratch_shapes=[
            pltpu.VMEM((2 * N_JN, M_PER, NB), jnp.bfloat16),
            pltpu.SemaphoreType.DMA((N_T - 1, N_JN)),
            pltpu.SemaphoreType.DMA((N_T - 1, N_JN)),
            pltpu.SemaphoreType.DMA,
        ],
        compiler_params=pltpu.CompilerParams(collective_id=0),
    )(x, w_mat)


# device time: 146557 ns/iter; 1.0055x vs baseline; 1.0055x over previous
import jax
import jax.numpy as jnp
from jax import lax
from jax.experimental import pallas as pl
from jax.experimental.pallas import tpu as pltpu

N_DEV = 4
M_PER = 1024
K = 4096
N_PER = 2048
NB = 256
N_T = 4
N_JN = N_PER // NB
N_BLK = N_T * N_JN


def kernel(x, w_mat):
    x = x.astype(jnp.bfloat16)

    def body(x_ref, w_ref, out_ref, wbf, send_buf, send_sems,
             recv_sems, local_sem):
        n = pl.program_id(0)
        my = lax.axis_index("i")

        @pl.when(n == 0)
        def _():
            bar = pltpu.get_barrier_semaphore()
            for off in (1, 2, 3):
                pl.semaphore_signal(
                    bar, inc=1,
                    device_id=((my + off) % N_DEV,),
                    device_id_type=pl.DeviceIdType.MESH,
                )
            pl.semaphore_wait(bar, 3)

        @pl.when(n < N_BLK)
        def _():
            wbf[n % 2] = w_ref[...].astype(jnp.bfloat16)

        def my_dst(jj):
            return out_ref.at[pl.ds(my * M_PER, M_PER), pl.ds(jj * NB, NB)]

        @pl.when(n >= 1)
        def _():
            m = n - 1
            tt = m // N_JN
            jj = m % N_JN
            slot = (tt % 2) * N_JN + jj

            part = jnp.dot(
                x_ref[...],
                wbf[m % 2],
                preferred_element_type=jnp.float32,
            )

            @pl.when(tt >= 2)
            def _():
                pltpu.make_async_remote_copy(
                    src_ref=send_buf.at[slot],
                    dst_ref=my_dst(jj),
                    send_sem=send_sems.at[tt - 2, jj],
                    recv_sem=recv_sems.at[0, 0],
                    device_id=(my,),
                    device_id_type=pl.DeviceIdType.MESH,
                ).wait_send()

            send_buf[slot] = (part * jax.nn.sigmoid(part)).astype(jnp.bfloat16)

            @pl.when(tt < N_T - 1)
            def _():
                d = (my + 1 + tt) % N_DEV
                rdma = pltpu.make_async_remote_copy(
                    src_ref=send_buf.at[slot],
                    dst_ref=my_dst(jj),
                    send_sem=send_sems.at[tt, jj],
                    recv_sem=recv_sems.at[tt, jj],
                    device_id=(d,),
                    device_id_type=pl.DeviceIdType.MESH,
                )
                rdma.start()

            @pl.when(tt == N_T - 1)
            def _():
                cp = pltpu.make_async_copy(send_buf.at[slot], my_dst(jj),
                                           local_sem)
                cp.start()

        @pl.when(n == N_BLK)
        def _():
            for jj in range(N_JN):
                pltpu.make_async_remote_copy(
                    src_ref=send_buf.at[jj],
                    dst_ref=my_dst(jj),
                    send_sem=send_sems.at[N_T - 2, jj],
                    recv_sem=recv_sems.at[0, 0],
                    device_id=(my,),
                    device_id_type=pl.DeviceIdType.MESH,
                ).wait_send()
            for tt in range(N_T - 1):
                for jj in range(N_JN):
                    pltpu.make_async_remote_copy(
                        src_ref=send_buf.at[jj],
                        dst_ref=my_dst(jj),
                        send_sem=send_sems.at[tt, jj],
                        recv_sem=recv_sems.at[tt, jj],
                        device_id=(my,),
                        device_id_type=pl.DeviceIdType.MESH,
                    ).wait_recv()
            for jj in range(N_JN):
                pltpu.make_async_copy(
                    send_buf.at[N_JN + jj], my_dst(jj), local_sem,
                ).wait()

    def w_imap(n):
        nc = jnp.minimum(n, N_BLK - 1)
        d = (lax.axis_index("i") + 1 + nc // N_JN) % N_DEV
        return (0, d * N_JN + nc % N_JN)

    return pl.pallas_call(
        body,
        grid=(N_BLK + 1,),
        out_shape=jax.ShapeDtypeStruct((N_DEV * M_PER, N_PER), jnp.bfloat16),
        in_specs=[
            pl.BlockSpec((M_PER, K), lambda n: (0, 0)),
            pl.BlockSpec((K, NB), w_imap),
        ],
        out_specs=pl.BlockSpec(memory_space=pl.ANY),
        scratch_shapes=[
            pltpu.VMEM((2, K, NB), jnp.bfloat16),
            pltpu.VMEM((2 * N_JN, M_PER, NB), jnp.bfloat16),
            pltpu.SemaphoreType.DMA((N_T - 1, N_JN)),
            pltpu.SemaphoreType.DMA((N_T - 1, N_JN)),
            pltpu.SemaphoreType.DMA,
        ],
        compiler_params=pltpu.CompilerParams(collective_id=0),
    )(x, w_mat)
